# baseline (speedup 1.0000x reference)
import jax
import jax.numpy as jnp
from jax.experimental import pallas as pl
from jax.experimental.pallas import tpu as pltpu

_BT = 2048


def _weights_body(p_ref, m_ref, w_ref, tpe_ref):
    m = m_ref[...]
    w_ref[...] = jnp.sum(p_ref[...] * m, axis=1, keepdims=True)
    tpe_ref[...] = jnp.sum(m, axis=0, keepdims=True)


def _scale_body(hs_ref, w_ref, out_ref):
    out_ref[...] = hs_ref[...] * w_ref[...]


def kernel(hidden_states, probs, routing_map):
    hidden_shape = hidden_states.shape
    H = hidden_shape[-1]
    T, E = probs.shape
    hs = hidden_states.reshape(T, H)
    mask = routing_map.astype(jnp.float32)

    w, tpe = pl.pallas_call(
        _weights_body,
        out_shape=[
            jax.ShapeDtypeStruct((T, 1), jnp.float32),
            jax.ShapeDtypeStruct((1, E), jnp.float32),
        ],
    )(probs, mask)

    out = pl.pallas_call(
        _scale_body,
        grid=(T // _BT,),
        in_specs=[
            pl.BlockSpec((_BT, H), lambda i: (i, 0)),
            pl.BlockSpec((_BT, 1), lambda i: (i, 0)),
        ],
        out_specs=pl.BlockSpec((_BT, H), lambda i: (i, 0)),
        out_shape=jax.ShapeDtypeStruct((T, H), hs.dtype),
        compiler_params=pltpu.CompilerParams(
            dimension_semantics=("parallel",),
        ),
    )(hs, w)

    tokens_per_expert = tpe.reshape(E).astype(jnp.int32)
    return out.reshape(hidden_shape), tokens_per_expert
